# baseline (speedup 1.0000x reference)
import functools

import jax
import jax.numpy as jnp
from jax import lax
from jax.experimental import pallas as pl
from jax.experimental.pallas import tpu as pltpu
from jax.experimental.pallas import tpu_sc as plsc

_N_CLS = 1024
_N_CTX = 16
_DIM = 768
_SEQ = 77
_NC = 1
_NS = 16
_NW = _NC * _NS
_CPW = _N_CLS // _NW
_MAIN = 48
_TAIL = _SEQ - 72


_mesh = plsc.VectorSubcoreMesh(core_axis_name="c", subcore_axis_name="s",
                               num_cores=1)


@functools.partial(
    pl.kernel,
    mesh=_mesh,
    out_type=jax.ShapeDtypeStruct((_N_CLS, _SEQ, _DIM), jnp.float32),
    scratch_types=[
        pltpu.VMEM((_SEQ,), jnp.int32),
        pltpu.VMEM((_SEQ,), jnp.int32),
        pltpu.VMEM((16,), jnp.int32),
        pltpu.VMEM((32,), jnp.int32),
        pltpu.VMEM((72, _DIM), jnp.float32),
        pltpu.VMEM((72, _DIM), jnp.float32),
        pltpu.VMEM((_TAIL, _DIM), jnp.float32),
        pltpu.VMEM((16, _DIM), jnp.float32),
        pltpu.SemaphoreType.DMA,
        pltpu.SemaphoreType.DMA,
        pltpu.SemaphoreType.DMA,
        pltpu.SemaphoreType.DMA,
        pltpu.SemaphoreType.DMA,
        pltpu.SemaphoreType.DMA,
        pltpu.SemaphoreType.DMA,
        pltpu.SemaphoreType.DMA,
    ],
    compiler_params=pltpu.CompilerParams(needs_layout_passes=False),
)
def _prompt_kernel(tok_hbm, table_hbm, ctx_hbm, out_hbm,
                   trow0_v, trow1_v, tidx_v, cidx_v, buf0_v, buf1_v,
                   tail_v, aux_v, st0, st1, sg0, sg1, sa, sw0, sw1, sw2):
    wid = lax.axis_index("s") * _NC + lax.axis_index("c")
    base_c = wid * _CPW
    trows = (trow0_v, trow1_v)
    bufs = (buf0_v, buf1_v)
    sts = (st0, st1)
    sgs = (sg0, sg1)
    sws = (sw0, sw1)

    i16 = lax.iota(jnp.int32, 16)
    cidx_v[pl.ds(0, 16)] = jnp.maximum(i16 - 1, 0)
    cidx_v[pl.ds(16, 16)] = jnp.full((16,), _N_CTX - 1, jnp.int32)
    pltpu.async_copy(ctx_hbm.at[cidx_v.at[pl.ds(0, 24)]],
                     buf0_v.at[pl.ds(0, 24)], sa).wait()
    pltpu.async_copy(ctx_hbm.at[cidx_v.at[pl.ds(0, 24)]],
                     buf1_v.at[pl.ds(0, 24)], sa).wait()

    tpos = jnp.where(i16 == 0, 0,
                     jnp.where(i16 < 8, i16 + 16,
                               jnp.minimum(i16 + 64, _SEQ - 1)))

    def trow_start(ci, b):
        pltpu.async_copy(tok_hbm.at[base_c + ci], trows[b], sts[b])

    def trow_wait(b):
        pltpu.make_async_copy(tok_hbm.at[base_c], trows[b], sts[b]).wait()

    def main_start(ci, b):
        pltpu.async_copy(table_hbm.at[trows[b].at[pl.ds(24, _MAIN)]],
                         bufs[b].at[pl.ds(24, _MAIN)], sgs[b])

    def main_wait(b):
        pltpu.make_async_copy(table_hbm.at[trows[b].at[pl.ds(24, _MAIN)]],
                              bufs[b].at[pl.ds(24, _MAIN)], sgs[b]).wait()

    def aux_start(b):
        tvals = plsc.load_gather(trows[b], [tpos])
        plsc.store_scatter(tidx_v, [i16], tvals, mask=i16 < 16)
        pltpu.async_copy(table_hbm.at[tidx_v], aux_v, sa)

    def aux_wait():
        pltpu.make_async_copy(table_hbm.at[tidx_v], aux_v, sa).wait()

    def fixups(b):
        for k in range(_DIM // 16):
            sl = pl.ds(16 * k, 16)
            bufs[b][0, sl] = aux_v[0, sl]
            for r in range(7):
                bufs[b][17 + r, sl] = aux_v[1 + r, sl]
            for t in range(_TAIL):
                tail_v[t, sl] = aux_v[8 + t, sl]

    def write_start(ci, b):
        pltpu.async_copy(bufs[b], out_hbm.at[base_c + ci, pl.ds(0, 72)],
                         sws[b])
        pltpu.async_copy(tail_v, out_hbm.at[base_c + ci, pl.ds(72, _TAIL)],
                         sw2)

    def w1_wait(b):
        pltpu.make_async_copy(bufs[b], out_hbm.at[base_c, pl.ds(0, 72)],
                              sws[b]).wait()

    def w2_wait():
        pltpu.make_async_copy(tail_v, out_hbm.at[base_c, pl.ds(72, _TAIL)],
                              sw2).wait()

    def iteration(ci, b, first, last):
        trow_wait(b)
        if not first:
            w1_wait(b)
        main_start(ci, b)
        main_wait(1 - b)
        aux_wait()
        if not first:
            w2_wait()
        fixups(1 - b)
        aux_start(b)
        write_start(ci - 1, 1 - b)
        if not last:
            trow_start(ci + 1, 1 - b)

    pltpu.sync_copy(tok_hbm.at[base_c], trow0_v)
    main_start(0, 0)
    aux_start(0)
    trow_start(1, 1)
    iteration(1, 1, True, False)

    def body(g, carry):
        ci = 2 * g + 2
        iteration(ci, 0, False, False)
        iteration(ci + 1, 1, False, False)
        return carry

    lax.fori_loop(0, (_CPW - 4) // 2, body, 0)
    iteration(_CPW - 2, 0, False, False)
    iteration(_CPW - 1, 1, False, True)

    main_wait(1)
    aux_wait()
    w2_wait()
    fixups(1)
    write_start(_CPW - 1, 1)
    w1_wait(0)
    w1_wait(1)
    w2_wait()


def kernel(tokenized_prompts, token_embedding, ctx):
    return _prompt_kernel(tokenized_prompts, token_embedding,
                          ctx.astype(jnp.float32))


# speedup vs baseline: 1.1009x; 1.1009x over previous
import functools

import jax
import jax.numpy as jnp
from jax import lax
from jax.experimental import pallas as pl
from jax.experimental.pallas import tpu as pltpu
from jax.experimental.pallas import tpu_sc as plsc

_N_CLS = 1024
_N_CTX = 16
_DIM = 768
_SEQ = 77
_NC = 2
_NS = 16
_NW = _NC * _NS
_CPW = _N_CLS // _NW
_MAIN = 48
_TAIL = _SEQ - 72


_mesh = plsc.VectorSubcoreMesh(core_axis_name="c", subcore_axis_name="s")


@functools.partial(
    pl.kernel,
    mesh=_mesh,
    out_type=jax.ShapeDtypeStruct((_N_CLS, _SEQ, _DIM), jnp.float32),
    scratch_types=[
        pltpu.VMEM((_SEQ,), jnp.int32),
        pltpu.VMEM((_SEQ,), jnp.int32),
        pltpu.VMEM((16,), jnp.int32),
        pltpu.VMEM((32,), jnp.int32),
        pltpu.VMEM((72, _DIM), jnp.float32),
        pltpu.VMEM((72, _DIM), jnp.float32),
        pltpu.VMEM((_TAIL, _DIM), jnp.float32),
        pltpu.VMEM((16, _DIM), jnp.float32),
        pltpu.SemaphoreType.DMA,
        pltpu.SemaphoreType.DMA,
        pltpu.SemaphoreType.DMA,
        pltpu.SemaphoreType.DMA,
        pltpu.SemaphoreType.DMA,
        pltpu.SemaphoreType.DMA,
        pltpu.SemaphoreType.DMA,
        pltpu.SemaphoreType.DMA,
    ],
    compiler_params=pltpu.CompilerParams(needs_layout_passes=False,
                                         disable_bounds_checks=True,
                                         disable_semaphore_checks=True),
)
def _prompt_kernel(tok_hbm, table_hbm, ctx_hbm, out_hbm,
                   trow0_v, trow1_v, tidx_v, cidx_v, buf0_v, buf1_v,
                   tail_v, aux_v, st0, st1, sg0, sg1, sa, sw0, sw1, sw2):
    wid = lax.axis_index("s") * _NC + lax.axis_index("c")
    base_c = wid * _CPW
    trows = (trow0_v, trow1_v)
    bufs = (buf0_v, buf1_v)
    sts = (st0, st1)
    sgs = (sg0, sg1)
    sws = (sw0, sw1)

    i16 = lax.iota(jnp.int32, 16)
    cidx_v[pl.ds(0, 16)] = jnp.maximum(i16 - 1, 0)
    cidx_v[pl.ds(16, 16)] = jnp.full((16,), _N_CTX - 1, jnp.int32)
    pltpu.async_copy(ctx_hbm.at[cidx_v.at[pl.ds(0, 24)]],
                     buf0_v.at[pl.ds(0, 24)], sa).wait()
    pltpu.async_copy(ctx_hbm.at[cidx_v.at[pl.ds(0, 24)]],
                     buf1_v.at[pl.ds(0, 24)], sa).wait()

    tpos = jnp.where(i16 == 0, 0,
                     jnp.where(i16 < 8, i16 + 16,
                               jnp.minimum(i16 + 64, _SEQ - 1)))

    def trow_start(ci, b):
        pltpu.async_copy(tok_hbm.at[base_c + ci], trows[b], sts[b])

    def trow_wait(b):
        pltpu.make_async_copy(tok_hbm.at[base_c], trows[b], sts[b]).wait()

    def main_start(ci, b):
        pltpu.async_copy(table_hbm.at[trows[b].at[pl.ds(24, _MAIN)]],
                         bufs[b].at[pl.ds(24, _MAIN)], sgs[b])

    def main_wait(b):
        pltpu.make_async_copy(table_hbm.at[trows[b].at[pl.ds(24, _MAIN)]],
                              bufs[b].at[pl.ds(24, _MAIN)], sgs[b]).wait()

    def aux_start(b):
        tvals = plsc.load_gather(trows[b], [tpos])
        plsc.store_scatter(tidx_v, [i16], tvals, mask=i16 < 16)
        pltpu.async_copy(table_hbm.at[tidx_v], aux_v, sa)

    def aux_wait():
        pltpu.make_async_copy(table_hbm.at[tidx_v], aux_v, sa).wait()

    def fixups(b):
        for k in range(_DIM // 16):
            sl = pl.ds(16 * k, 16)
            bufs[b][0, sl] = aux_v[0, sl]
            for r in range(7):
                bufs[b][17 + r, sl] = aux_v[1 + r, sl]
            for t in range(_TAIL):
                tail_v[t, sl] = aux_v[8 + t, sl]

    def write_start(ci, b):
        pltpu.async_copy(bufs[b], out_hbm.at[base_c + ci, pl.ds(0, 72)],
                         sws[b])
        pltpu.async_copy(tail_v, out_hbm.at[base_c + ci, pl.ds(72, _TAIL)],
                         sw2)

    def w1_wait(b):
        pltpu.make_async_copy(bufs[b], out_hbm.at[base_c, pl.ds(0, 72)],
                              sws[b]).wait()

    def w2_wait():
        pltpu.make_async_copy(tail_v, out_hbm.at[base_c, pl.ds(72, _TAIL)],
                              sw2).wait()

    def iteration(ci, b, first, last):
        trow_wait(b)
        if not first:
            w1_wait(b)
        main_start(ci, b)
        main_wait(1 - b)
        aux_wait()
        if not first:
            w2_wait()
        fixups(1 - b)
        aux_start(b)
        write_start(ci - 1, 1 - b)
        if not last:
            trow_start(ci + 1, 1 - b)

    pltpu.sync_copy(tok_hbm.at[base_c], trow0_v)
    main_start(0, 0)
    aux_start(0)
    trow_start(1, 1)
    iteration(1, 1, True, False)

    def body(g, carry):
        ci = 2 * g + 2
        iteration(ci, 0, False, False)
        iteration(ci + 1, 1, False, False)
        return carry

    lax.fori_loop(0, (_CPW - 4) // 2, body, 0)
    iteration(_CPW - 2, 0, False, False)
    iteration(_CPW - 1, 1, False, True)

    main_wait(1)
    aux_wait()
    w2_wait()
    fixups(1)
    write_start(_CPW - 1, 1)
    w1_wait(0)
    w1_wait(1)
    w2_wait()


def kernel(tokenized_prompts, token_embedding, ctx):
    return _prompt_kernel(tokenized_prompts, token_embedding,
                          ctx.astype(jnp.float32))
